# Initial kernel scaffold; baseline (speedup 1.0000x reference)
import jax
import jax.numpy as jnp
from jax.experimental import pallas as pl


def kernel(feats, edges, W, b):
    raise NotImplementedError("write your pallas kernel here")


# speedup vs baseline: 15.7702x; 15.7702x over previous
import functools

import jax
import jax.numpy as jnp
from jax import lax
from jax.experimental import pallas as pl
from jax.experimental.pallas import tpu as pltpu
from jax.experimental.pallas import tpu_sc as plsc

N_NODES = 10000
D = 128
E = 320000
NC, NS = 2, 16
N_PAD = 10240
RPT = N_PAD // NS
CH = 80
EPT = E // (NC * NS)
NCHUNK = EPT // CH

_MESH = plsc.VectorSubcoreMesh(
    core_axis_name="c", subcore_axis_name="s", num_cores=NC, num_subcores=NS
)


@functools.partial(
    pl.kernel,
    out_type=jax.ShapeDtypeStruct((NC, N_PAD), jnp.float32),
    mesh=_MESH,
    scratch_types=[
        pltpu.VMEM((CH,), jnp.int32),
        pltpu.VMEM((CH,), jnp.float32),
        pltpu.VMEM((RPT,), jnp.float32),
        pltpu.VMEM_SHARED((N_PAD,), jnp.float32),
    ],
)
def _deg_kernel(dst_hbm, ones_hbm, zrow_hbm, degp_hbm, idx_v, ones_v, buf_v,
                deg_sh):
    c = lax.axis_index("c")
    s = lax.axis_index("s")
    pltpu.sync_copy(ones_hbm, ones_v)
    pltpu.sync_copy(zrow_hbm, buf_v)
    row0 = pl.multiple_of(s * RPT, 8)
    pltpu.sync_copy(buf_v, deg_sh.at[pl.ds(row0, RPT)])
    plsc.subcore_barrier()

    base = (c * NS + s) * EPT

    def body(i, carry):
        off = pl.multiple_of(base + i * CH, 8)
        pltpu.sync_copy(dst_hbm.at[pl.ds(off, CH)], idx_v)
        pltpu.sync_copy(ones_v, deg_sh.at[idx_v], add=True)
        return carry

    lax.fori_loop(0, NCHUNK, body, 0)
    plsc.subcore_barrier()
    pltpu.sync_copy(deg_sh.at[pl.ds(row0, RPT)], buf_v)
    pltpu.sync_copy(buf_v, degp_hbm.at[c, pl.ds(row0, RPT)])


@functools.partial(
    pl.kernel,
    out_type=jax.ShapeDtypeStruct((NC, N_PAD, D), jnp.float32),
    mesh=_MESH,
    scratch_types=[
        pltpu.VMEM((CH,), jnp.int32),
        pltpu.VMEM((CH,), jnp.int32),
        pltpu.VMEM((CH, D), jnp.float32),
        pltpu.VMEM((CH, D), jnp.float32),
        pltpu.VMEM_SHARED((N_PAD, D), jnp.float32),
        pltpu.SemaphoreType.DMA,
    ],
)
def _sweep_kernel(src_hbm, dst_hbm, xt_hbm, zrows_hbm, q_hbm, src_v, dst_v,
                  rows_v, zbuf_v, out_sh, sem):
    c = lax.axis_index("c")
    s = lax.axis_index("s")
    pltpu.sync_copy(zrows_hbm, zbuf_v)
    for j in range(RPT // CH):
        r0 = pl.multiple_of(s * RPT + j * CH, 8)
        pltpu.sync_copy(zbuf_v, out_sh.at[pl.ds(r0, CH)])
    plsc.subcore_barrier()

    base = (c * NS + s) * EPT

    def body(i, carry):
        off = pl.multiple_of(base + i * CH, 8)
        pltpu.sync_copy(src_hbm.at[pl.ds(off, CH)], src_v)
        pltpu.sync_copy(dst_hbm.at[pl.ds(off, CH)], dst_v)
        pltpu.async_copy(xt_hbm.at[src_v], rows_v, sem).wait()
        pltpu.sync_copy(rows_v, out_sh.at[dst_v], add=True)
        return carry

    lax.fori_loop(0, NCHUNK, body, 0)
    plsc.subcore_barrier()
    for j in range(RPT // CH):
        r0 = pl.multiple_of(s * RPT + j * CH, 8)
        pltpu.sync_copy(out_sh.at[pl.ds(r0, CH)], rows_v)
        pltpu.sync_copy(rows_v, q_hbm.at[c, pl.ds(r0, CH)])


def _mm_body(feats_ref, w_ref, d0_ref, d1_ref, xt_ref, dinv_ref):
    deg = d0_ref[...] + d1_ref[...]
    good = deg > 0.0
    dinv = jnp.where(good, lax.rsqrt(jnp.where(good, deg, 1.0)), 0.0)
    x = jnp.dot(feats_ref[...], w_ref[...], preferred_element_type=jnp.float32)
    xt_ref[...] = x * dinv
    dinv_ref[...] = dinv


def _fin_body(q_ref, dinv_ref, b_ref, o_ref):
    acc = q_ref[0, :N_NODES, :] + q_ref[1, :N_NODES, :]
    o_ref[...] = acc * dinv_ref[...] + b_ref[...]


def kernel(feats, edges, W, b):
    src = edges[0].astype(jnp.int32)
    dst = edges[1].astype(jnp.int32)
    ones_c = jnp.ones((CH,), jnp.float32)
    zrow = jnp.zeros((RPT,), jnp.float32)
    zrows = jnp.zeros((CH, D), jnp.float32)

    degp = _deg_kernel(dst, ones_c, zrow)
    deg0 = degp[0, :N_NODES, None]
    deg1 = degp[1, :N_NODES, None]

    xt, dinv = pl.pallas_call(
        _mm_body,
        out_shape=(
            jax.ShapeDtypeStruct((N_NODES, D), jnp.float32),
            jax.ShapeDtypeStruct((N_NODES, 1), jnp.float32),
        ),
    )(feats, W, deg0, deg1)

    q = _sweep_kernel(src, dst, xt, zrows)

    out = pl.pallas_call(
        _fin_body,
        out_shape=jax.ShapeDtypeStruct((N_NODES, D), jnp.float32),
    )(q, dinv, b.reshape(1, D))
    return out
